# baseline (speedup 1.0000x reference)
import jax
import jax.numpy as jnp
from jax import lax
from jax.experimental import pallas as pl
from jax.experimental.pallas import tpu as pltpu, tpu_sc as plsc

_NC, _NS = 2, 16
_NW = _NC * _NS
_TR = 4
_BPU = 2
_NB = 4
_LEAD = 2


def _sc_kernel(x, pos_table):
    b, t, d = x.shape
    rpu = _TR * _BPU
    rows_w = t // _NW
    chunks = rows_w // _TR
    upc = b // _BPU
    units = chunks * upc
    upg = 2 * upc
    groups = units // upg

    def body(x_hbm, pos_hbm, out_hbm, pos0, pos1, xw0, xw1, xw2, xw3,
             pidx0, pidx1, xidx0, xidx1, xidx2, xidx3,
             psem0, psem1, isem0, isem1, isem2, isem3,
             osem0, osem1, osem2, osem3):
        pos_v = (pos0, pos1)
        xw = (xw0, xw1, xw2, xw3)
        pidx = (pidx0, pidx1)
        xidx = (xidx0, xidx1, xidx2, xidx3)
        psem = (psem0, psem1)
        isem = (isem0, isem1, isem2, isem3)
        osem = (osem0, osem1, osem2, osem3)

        wid = lax.axis_index("s") * _NC + lax.axis_index("c")
        base = wid * rows_w
        iota = lax.iota(jnp.int32, 16)
        unit_pat = (iota & (_TR - 1)) + (iota >> 2) * t

        def stage_pos(c, p):
            pidx[p][...] = (base + c * _TR) + iota
            pltpu.async_copy(pos_hbm.at[pidx[p].at[pl.ds(0, _TR)]],
                             pos_v[p], psem[p])

        def wait_pos(p):
            pltpu.make_async_copy(pos_hbm.at[pidx[p].at[pl.ds(0, _TR)]],
                                  pos_v[p], psem[p]).wait()

        def stage_x(c, h, k):
            xidx[k][...] = ((h * _BPU) * t + base + c * _TR) + unit_pat
            pltpu.async_copy(x_hbm.at[xidx[k].at[pl.ds(0, rpu)]],
                             xw[k], isem[k])

        def wait_x(k):
            pltpu.make_async_copy(x_hbm.at[xidx[k].at[pl.ds(0, rpu)]],
                                  xw[k], isem[k]).wait()

        def store_out(k):
            pltpu.async_copy(xw[k], out_hbm.at[xidx[k].at[pl.ds(0, rpu)]],
                             osem[k])

        def drain_out(k):
            pltpu.make_async_copy(xw[k], out_hbm.at[xidx[k].at[pl.ds(0, rpu)]],
                                  osem[k]).wait()

        stage_pos(0, 0)
        for u in range(_LEAD):
            stage_x(0, u, u)

        @pl.loop(0, groups)
        def _(g):
            for uu in range(upg):
                cc, h = divmod(uu, upc)
                k = uu % _NB
                c = 2 * g + cc

                if uu == 0:
                    stage_pos(c + 1, 1)
                    wait_pos(0)
                if uu == upc:
                    @pl.when(g < groups - 1)
                    def _():
                        stage_pos(c + 1, 0)
                    wait_pos(1)

                wait_x(k)

                for r in range(rpu):
                    @plsc.parallel_loop(0, d, step=16, unroll=8)
                    def _(o):
                        plsc.addupdate(xw[k].at[r, pl.ds(o, 16)],
                                       pos_v[cc][r % _TR, pl.ds(o, 16)])

                store_out(k)

                vcc, vh = divmod(uu + _LEAD, upc)
                vk = (uu + _LEAD) % _NB
                vc = 2 * g + vcc

                def _service(vc=vc, vh=vh, vk=vk):
                    drain_out(vk)
                    stage_x(vc, vh, vk)

                if uu + _LEAD < upg:
                    if uu < _LEAD:
                        pl.when(g > 0)(lambda vk=vk: drain_out(vk))
                        stage_x(vc, vh, vk)
                    else:
                        _service()
                else:
                    pl.when(g < groups - 1)(_service)

        for u in range(units - 2 * _LEAD, units):
            drain_out(u % _NB)

    out = pl.kernel(
        body,
        out_type=jax.ShapeDtypeStruct((b * t, d), x.dtype),
        mesh=plsc.VectorSubcoreMesh(core_axis_name="c", subcore_axis_name="s"),
        scratch_types=(
            [pltpu.VMEM((_TR, d), jnp.float32)] * 2
            + [pltpu.VMEM((rpu, d), jnp.float32)] * _NB
            + [pltpu.VMEM((16,), jnp.int32)] * (2 + _NB)
            + [pltpu.SemaphoreType.DMA] * (2 + 2 * _NB)
        ),
    )(x.reshape(b * t, d), pos_table[:t])
    return out.reshape(b, t, d)


def kernel(x, pos_table):
    return _sc_kernel(x, pos_table)


# speedup vs baseline: 1.0554x; 1.0554x over previous
import jax
import jax.numpy as jnp
from jax import lax
from jax.experimental import pallas as pl
from jax.experimental.pallas import tpu as pltpu, tpu_sc as plsc

_NC, _NS = 2, 16
_NW = _NC * _NS
_R = 4
_NB = 8
_LEAD = 6


def _sc_kernel(x, pos_table):
    b, t, d = x.shape
    rows_w = t // _NW
    chunks = rows_w // _R
    upg = 2 * b
    groups = (chunks * b) // upg

    def body(x_hbm, pos_hbm, out_hbm, pos0, pos1,
             xw0, xw1, xw2, xw3, xw4, xw5, xw6, xw7,
             pidx0, pidx1,
             xidx0, xidx1, xidx2, xidx3, xidx4, xidx5, xidx6, xidx7,
             psem0, psem1,
             isem0, isem1, isem2, isem3, isem4, isem5, isem6, isem7,
             osem0, osem1, osem2, osem3, osem4, osem5, osem6, osem7):
        pos_v = (pos0, pos1)
        xw = (xw0, xw1, xw2, xw3, xw4, xw5, xw6, xw7)
        pidx = (pidx0, pidx1)
        xidx = (xidx0, xidx1, xidx2, xidx3, xidx4, xidx5, xidx6, xidx7)
        psem = (psem0, psem1)
        isem = (isem0, isem1, isem2, isem3, isem4, isem5, isem6, isem7)
        osem = (osem0, osem1, osem2, osem3, osem4, osem5, osem6, osem7)

        wid = lax.axis_index("s") * _NC + lax.axis_index("c")
        base = wid * rows_w
        iota = lax.iota(jnp.int32, 16)

        def pos_rows(c):
            return base + c * _R

        def x_rows(c, bb):
            return bb * t + base + c * _R

        def stage_pos(c, p):
            pidx[p][...] = pos_rows(c) + iota
            pltpu.async_copy(pos_hbm.at[pidx[p].at[pl.ds(0, _R)]],
                             pos_v[p], psem[p])

        def wait_pos(p):
            pltpu.make_async_copy(pos_hbm.at[pidx[p].at[pl.ds(0, _R)]],
                                  pos_v[p], psem[p]).wait()

        def stage_x(c, bb, k):
            xidx[k][...] = x_rows(c, bb) + iota
            pltpu.async_copy(x_hbm.at[xidx[k].at[pl.ds(0, _R)]],
                             xw[k], isem[k])

        def wait_x(k):
            pltpu.make_async_copy(x_hbm.at[xidx[k].at[pl.ds(0, _R)]],
                                  xw[k], isem[k]).wait()

        def store_out(k):
            pltpu.async_copy(xw[k], out_hbm.at[xidx[k].at[pl.ds(0, _R)]],
                             osem[k])

        def drain_out(k):
            pltpu.make_async_copy(xw[k], out_hbm.at[xidx[k].at[pl.ds(0, _R)]],
                                  osem[k]).wait()

        stage_pos(0, 0)
        for u in range(_LEAD):
            stage_x(u // b, u % b, u)

        @pl.loop(0, groups)
        def _(g):
            for uu in range(upg):
                cc, bb = divmod(uu, b)
                k = uu % _NB
                c = 2 * g + cc

                if uu == 0:
                    stage_pos(c + 1, 1)
                    wait_pos(0)
                if uu == b:
                    @pl.when(g < groups - 1)
                    def _():
                        stage_pos(c + 1, 0)
                    wait_pos(1)

                wait_x(k)

                for r in range(_R):
                    @plsc.parallel_loop(0, d, step=16, unroll=8)
                    def _(o):
                        plsc.addupdate(xw[k].at[r, pl.ds(o, 16)],
                                       pos_v[cc][r, pl.ds(o, 16)])

                store_out(k)

                vcc, vbb = divmod(uu + _LEAD, b)
                vk = (uu + _LEAD) % _NB
                vc = 2 * g + vcc

                def _service(vc=vc, vbb=vbb, vk=vk):
                    drain_out(vk)
                    stage_x(vc, vbb, vk)

                if uu + _LEAD < upg:
                    if uu < _LEAD:
                        pl.when(g > 0)(lambda vk=vk: drain_out(vk))
                        stage_x(vc, vbb, vk)
                    else:
                        _service()
                else:
                    pl.when(g < groups - 1)(_service)

        for u in range(chunks * b - _NB, chunks * b):
            k = u % _NB
            drain_out(k)

    out = pl.kernel(
        body,
        out_type=jax.ShapeDtypeStruct((b * t, d), x.dtype),
        mesh=plsc.VectorSubcoreMesh(core_axis_name="c", subcore_axis_name="s"),
        scratch_types=(
            [pltpu.VMEM((_R, d), jnp.float32)] * (2 + _NB)
            + [pltpu.VMEM((16,), jnp.int32)] * (2 + _NB)
            + [pltpu.SemaphoreType.DMA] * (2 + 2 * _NB)
        ),
    )(x.reshape(b * t, d), pos_table[:t])
    return out.reshape(b, t, d)


def kernel(x, pos_table):
    return _sc_kernel(x, pos_table)
